# baseline (speedup 1.0000x reference)
import functools
import jax
import jax.numpy as jnp
from jax import lax
from jax.experimental import pallas as pl
from jax.experimental.pallas import tpu as pltpu
from jax.experimental.pallas import tpu_sc as plsc

B, L, D = 1024, 200, 64
VOCAB = 100000
PADDING_IDX = 1
LN_EPS = 1e-6

NC, NS = 2, 16
NW = NC * NS
TOKENS = B * L
TPW = TOKENS // NW
CHUNK = 128
NCHUNK = TPW // CHUNK
NPAIR = NCHUNK // 2
PSEG_L = L - 8 + CHUNK


def _rsqrt(x):
    i = lax.bitcast_convert_type(x, jnp.int32)
    i = jnp.int32(0x5F3759DF) - lax.shift_right_arithmetic(i, 1)
    y = lax.bitcast_convert_type(i, jnp.float32)
    for _ in range(2):
        y = y * (1.5 - 0.5 * x * y * y)
    return y


def _allsum(v, perms):
    for p in perms:
        v = v + jnp.take_along_axis(v, p, axis=0)
    return v


def _wid():
    return lax.axis_index("s") * NC + lax.axis_index("c")


def _body(x_hbm, seg_hbm, emot_hbm, word_hbm, pos_hbm, segt_hbm, emott_hbm,
          gamma_hbm, beta_hbm, out_hbm,
          xb, sbm, ebm, rowsA, rowsB, obufA, obufB, pseg2, emott, posb, segtb,
          gb, bb, gsemA, gsemB, osemA, osemB):
    wid = _wid()
    iota16 = lax.iota(jnp.int32, 16)
    perms = [lax.bitwise_xor(iota16, jnp.int32(k)) for k in (8, 4, 2, 1)]
    rowv = [iota16 + c * 16 for c in range(4)]
    base0 = wid * TPW

    pltpu.sync_copy(x_hbm.at[wid], xb)
    pltpu.sync_copy(seg_hbm.at[wid], sbm)
    pltpu.sync_copy(emot_hbm.at[wid], ebm)
    pltpu.sync_copy(pos_hbm, posb)
    pltpu.sync_copy(segt_hbm, segtb)
    pltpu.sync_copy(emott_hbm, emott)
    pltpu.sync_copy(gamma_hbm, gb)
    pltpu.sync_copy(beta_hbm, bb)

    gamma_v = [gb[pl.ds(c * 16, 16)] for c in range(4)]
    beta_v = [bb[pl.ds(c * 16, 16)] for c in range(4)]
    seg0 = [segtb[0, pl.ds(c * 16, 16)] for c in range(4)]
    seg1 = [segtb[1, pl.ds(c * 16, 16)] for c in range(4)]

    def gather(c, rows, sem):
        return pltpu.make_async_copy(word_hbm.at[xb.at[c]], rows, sem)

    def scatter(c, obuf, sem):
        tg0 = base0 + c * CHUNK
        l = lax.shift_right_logical(tg0, 10)
        b0 = pl.multiple_of(lax.bitwise_and(tg0, B - 1), CHUNK)
        dst = out_hbm.at[l, :, pl.ds(b0, CHUNK)]
        return pltpu.make_async_copy(obuf, dst, sem)

    def compute(ci, rows, obuf):
        zero16 = iota16 * 0
        l = lax.shift_right_logical(base0 + ci * CHUNK, 10)
        for c in range(4):
            p = posb[l, pl.ds(c * 16, 16)]
            pseg2[0, pl.ds(c * 16, 16)] = p + seg0[c]
            pseg2[1, pl.ds(c * 16, 16)] = p + seg1[c]

        def group(g, _):
            tbase = g * 16
            sv = sbm[ci, pl.ds(tbase, 16)]
            ev = ebm[ci, pl.ds(tbase, 16)]
            for half in range(2):
                sumv = jnp.full((16,), 0.0, jnp.float32)
                sqv = jnp.full((16,), 0.0, jnp.float32)
                hs = []
                for jj in range(8):
                    j = half * 8 + jj
                    t = tbase + j
                    ps_row = sv[j]
                    e_row = ev[j]
                    h = [rows[t, pl.ds(c * 16, 16)]
                         + pseg2[ps_row, pl.ds(c * 16, 16)]
                         + emott[e_row, pl.ds(c * 16, 16)] for c in range(4)]
                    ssum = _allsum(h[0] + h[1] + h[2] + h[3], perms)
                    qsum = _allsum(h[0] * h[0] + h[1] * h[1]
                                   + h[2] * h[2] + h[3] * h[3], perms)
                    mj = iota16 == jj
                    sumv = jnp.where(mj, ssum, sumv)
                    sqv = jnp.where(mj, qsum, sqv)
                    hs.append(h)
                meanv = sumv * (1.0 / D)
                varv = sqv * (1.0 / D) - meanv * meanv
                rstdv = _rsqrt(varv + LN_EPS)
                for jj in range(8):
                    t = tbase + half * 8 + jj
                    idxj = zero16 + jj
                    colv = zero16 + t
                    mean_b = jnp.take_along_axis(meanv, idxj, axis=0)
                    rstd_b = jnp.take_along_axis(rstdv, idxj, axis=0)
                    for c in range(4):
                        y = ((hs[jj][c] - mean_b) * rstd_b
                             * gamma_v[c] + beta_v[c])
                        plsc.store_scatter(obuf, [rowv[c], colv], y)
            return _

        lax.fori_loop(0, CHUNK // 16, group, None)

    gather(0, rowsA, gsemA).start()

    def pair(c2, _):
        a = 2 * c2
        b = a + 1
        gather(b, rowsB, gsemB).start()
        gather(a, rowsA, gsemA).wait()

        @pl.when(c2 > 0)
        def _w1():
            scatter(a - 2, obufA, osemA).wait()

        compute(a, rowsA, obufA)
        scatter(a, obufA, osemA).start()

        @pl.when(c2 < NPAIR - 1)
        def _g1():
            gather(a + 2, rowsA, gsemA).start()

        gather(b, rowsB, gsemB).wait()

        @pl.when(c2 > 0)
        def _w2():
            scatter(b - 2, obufB, osemB).wait()

        compute(b, rowsB, obufB)
        scatter(b, obufB, osemB).start()
        return _

    lax.fori_loop(0, NPAIR, pair, None)
    scatter(NCHUNK - 2, obufA, osemA).wait()
    scatter(NCHUNK - 1, obufB, osemB).wait()


@jax.jit
def _embed_ln(xf, sf, ef, word_table, pos_slice, seg_table, emot_table,
              gamma, beta):
    mesh = plsc.VectorSubcoreMesh(core_axis_name="c", subcore_axis_name="s",
                                  num_cores=NC, num_subcores=NS)
    return pl.kernel(
        _body,
        out_type=jax.ShapeDtypeStruct((L, D, B), jnp.float32),
        mesh=mesh,
        compiler_params=pltpu.CompilerParams(use_tc_tiling_on_sc=False,
                                             needs_layout_passes=False),
        scratch_types=[
            pltpu.VMEM((NCHUNK, CHUNK), jnp.int32),
            pltpu.VMEM((NCHUNK, CHUNK), jnp.int32),
            pltpu.VMEM((NCHUNK, CHUNK), jnp.int32),
            pltpu.VMEM((CHUNK, D), jnp.float32),
            pltpu.VMEM((CHUNK, D), jnp.float32),
            pltpu.VMEM((D, CHUNK), jnp.float32),
            pltpu.VMEM((D, CHUNK), jnp.float32),
            pltpu.VMEM((2, D), jnp.float32),
            pltpu.VMEM((41, D), jnp.float32),
            pltpu.VMEM((L, D), jnp.float32),
            pltpu.VMEM((2, D), jnp.float32),
            pltpu.VMEM((D,), jnp.float32),
            pltpu.VMEM((D,), jnp.float32),
            pltpu.SemaphoreType.DMA,
            pltpu.SemaphoreType.DMA,
            pltpu.SemaphoreType.DMA,
            pltpu.SemaphoreType.DMA,
        ],
    )(xf, sf, ef, word_table, pos_slice, seg_table, emot_table, gamma, beta)


def kernel(x, seg, emot, training, word_table, pos_table, seg_table,
           emot_table, gamma, beta):
    xf = x.T.reshape(NW, NCHUNK, CHUNK).astype(jnp.int32)
    sf = seg.T.reshape(NW, NCHUNK, CHUNK).astype(jnp.int32)
    ef = emot.T.reshape(NW, NCHUNK, CHUNK).astype(jnp.int32)
    pos_slice = lax.slice(pos_table, (PADDING_IDX + 1, 0),
                          (L + PADDING_IDX + 1, D))
    out = _embed_ln(xf, sf, ef, word_table, pos_slice, seg_table,
                    emot_table, gamma, beta)
    return jnp.transpose(out, (2, 0, 1))


# speedup vs baseline: 1.5388x; 1.5388x over previous
import functools
import jax
import jax.numpy as jnp
from jax import lax
from jax.experimental import pallas as pl
from jax.experimental.pallas import tpu as pltpu
from jax.experimental.pallas import tpu_sc as plsc

B, L, D = 1024, 200, 64
VOCAB = 100000
PADDING_IDX = 1
LN_EPS = 1e-6

NC, NS = 2, 16
NW = NC * NS
TOKENS = B * L
TPW = TOKENS // NW
CHUNK = 128
NCHUNK = TPW // CHUNK
NPAIR = NCHUNK // 2
PSEG_L = L - 8 + CHUNK


def _rsqrt(x):
    i = lax.bitcast_convert_type(x, jnp.int32)
    i = jnp.int32(0x5F3759DF) - lax.shift_right_arithmetic(i, 1)
    y = lax.bitcast_convert_type(i, jnp.float32)
    for _ in range(2):
        y = y * (1.5 - 0.5 * x * y * y)
    return y


def _allsum(v, perms):
    for p in perms:
        v = v + jnp.take_along_axis(v, p, axis=0)
    return v


def _wid():
    return lax.axis_index("s") * NC + lax.axis_index("c")


def _body(x_hbm, seg_hbm, emot_hbm, word_hbm, pos_hbm, segt_hbm, emott_hbm,
          gamma_hbm, beta_hbm, out_hbm,
          xb, sbm, ebm, rowsA, rowsB, obufA, obufB, pseg2, emott, posb, segtb,
          gb, bb, gsemA, gsemB, osemA, osemB):
    wid = _wid()
    iota16 = lax.iota(jnp.int32, 16)
    perms = [lax.bitwise_xor(iota16, jnp.int32(k)) for k in (8, 4, 2, 1)]
    rowv = [iota16 + c * 16 for c in range(4)]
    base0 = wid * TPW

    pltpu.sync_copy(x_hbm.at[wid], xb)
    pltpu.sync_copy(seg_hbm.at[wid], sbm)
    pltpu.sync_copy(emot_hbm.at[wid], ebm)
    pltpu.sync_copy(pos_hbm, posb)
    pltpu.sync_copy(segt_hbm, segtb)
    pltpu.sync_copy(emott_hbm, emott)
    pltpu.sync_copy(gamma_hbm, gb)
    pltpu.sync_copy(beta_hbm, bb)

    gamma_v = [gb[pl.ds(c * 16, 16)] for c in range(4)]
    beta_v = [bb[pl.ds(c * 16, 16)] for c in range(4)]
    seg0 = [segtb[0, pl.ds(c * 16, 16)] for c in range(4)]
    seg1 = [segtb[1, pl.ds(c * 16, 16)] for c in range(4)]

    def gather(c, rows, sem):
        return pltpu.make_async_copy(word_hbm.at[xb.at[c]], rows, sem)

    def scatter(c, obuf, sem):
        tg0 = base0 + c * CHUNK
        l = lax.shift_right_logical(tg0, 10)
        b0 = pl.multiple_of(lax.bitwise_and(tg0, B - 1), CHUNK)
        dst = out_hbm.at[l, :, pl.ds(b0, CHUNK)]
        return pltpu.make_async_copy(obuf.at[:, pl.ds(0, CHUNK)], dst, sem)

    def compute(ci, rows, obuf):
        zero16 = iota16 * 0
        l = lax.shift_right_logical(base0 + ci * CHUNK, 10)
        for c in range(4):
            p = posb[l, pl.ds(c * 16, 16)]
            pseg2[0, pl.ds(c * 16, 16)] = p + seg0[c]
            pseg2[1, pl.ds(c * 16, 16)] = p + seg1[c]

        def group(g, _):
            tbase = g * 16
            sv = sbm[ci, pl.ds(tbase, 16)]
            ev = ebm[ci, pl.ds(tbase, 16)]
            for half in range(2):
                sumv = jnp.full((16,), 0.0, jnp.float32)
                sqv = jnp.full((16,), 0.0, jnp.float32)
                hs = []
                for jj in range(8):
                    j = half * 8 + jj
                    t = tbase + j
                    ps_row = sv[j]
                    e_row = ev[j]
                    h = [rows[t, pl.ds(c * 16, 16)]
                         + pseg2[ps_row, pl.ds(c * 16, 16)]
                         + emott[e_row, pl.ds(c * 16, 16)] for c in range(4)]
                    ssum = _allsum(h[0] + h[1] + h[2] + h[3], perms)
                    qsum = _allsum(h[0] * h[0] + h[1] * h[1]
                                   + h[2] * h[2] + h[3] * h[3], perms)
                    mj = iota16 == jj
                    sumv = jnp.where(mj, ssum, sumv)
                    sqv = jnp.where(mj, qsum, sqv)
                    hs.append(h)
                meanv = sumv * (1.0 / D)
                varv = sqv * (1.0 / D) - meanv * meanv
                rstdv = _rsqrt(varv + LN_EPS)
                for jj in range(8):
                    t = tbase + half * 8 + jj
                    idxj = zero16 + jj
                    colv = zero16 + t
                    mean_b = jnp.take_along_axis(meanv, idxj, axis=0)
                    rstd_b = jnp.take_along_axis(rstdv, idxj, axis=0)
                    for c in range(4):
                        y = ((hs[jj][c] - mean_b) * rstd_b
                             * gamma_v[c] + beta_v[c])
                        plsc.store_scatter(obuf, [rowv[c], colv], y)
            return _

        lax.fori_loop(0, CHUNK // 16, group, None)

    gather(0, rowsA, gsemA).start()

    def pair(c2, _):
        a = 2 * c2
        b = a + 1
        gather(b, rowsB, gsemB).start()
        gather(a, rowsA, gsemA).wait()

        @pl.when(c2 > 0)
        def _w1():
            scatter(a - 2, obufA, osemA).wait()

        compute(a, rowsA, obufA)
        scatter(a, obufA, osemA).start()

        @pl.when(c2 < NPAIR - 1)
        def _g1():
            gather(a + 2, rowsA, gsemA).start()

        gather(b, rowsB, gsemB).wait()

        @pl.when(c2 > 0)
        def _w2():
            scatter(b - 2, obufB, osemB).wait()

        compute(b, rowsB, obufB)
        scatter(b, obufB, osemB).start()
        return _

    lax.fori_loop(0, NPAIR, pair, None)
    scatter(NCHUNK - 2, obufA, osemA).wait()
    scatter(NCHUNK - 1, obufB, osemB).wait()


@jax.jit
def _embed_ln(xf, sf, ef, word_table, pos_slice, seg_table, emot_table,
              gamma, beta):
    mesh = plsc.VectorSubcoreMesh(core_axis_name="c", subcore_axis_name="s",
                                  num_cores=NC, num_subcores=NS)
    return pl.kernel(
        _body,
        out_type=jax.ShapeDtypeStruct((L, D, B), jnp.float32),
        mesh=mesh,
        compiler_params=pltpu.CompilerParams(use_tc_tiling_on_sc=False,
                                             needs_layout_passes=False),
        scratch_types=[
            pltpu.VMEM((NCHUNK, CHUNK), jnp.int32),
            pltpu.VMEM((NCHUNK, CHUNK), jnp.int32),
            pltpu.VMEM((NCHUNK, CHUNK), jnp.int32),
            pltpu.VMEM((CHUNK, D), jnp.float32),
            pltpu.VMEM((CHUNK, D), jnp.float32),
            pltpu.VMEM((D, CHUNK + 1), jnp.float32),
            pltpu.VMEM((D, CHUNK + 1), jnp.float32),
            pltpu.VMEM((2, D), jnp.float32),
            pltpu.VMEM((41, D), jnp.float32),
            pltpu.VMEM((L, D), jnp.float32),
            pltpu.VMEM((2, D), jnp.float32),
            pltpu.VMEM((D,), jnp.float32),
            pltpu.VMEM((D,), jnp.float32),
            pltpu.SemaphoreType.DMA,
            pltpu.SemaphoreType.DMA,
            pltpu.SemaphoreType.DMA,
            pltpu.SemaphoreType.DMA,
        ],
    )(xf, sf, ef, word_table, pos_slice, seg_table, emot_table, gamma, beta)


def kernel(x, seg, emot, training, word_table, pos_table, seg_table,
           emot_table, gamma, beta):
    xf = x.T.reshape(NW, NCHUNK, CHUNK).astype(jnp.int32)
    sf = seg.T.reshape(NW, NCHUNK, CHUNK).astype(jnp.int32)
    ef = emot.T.reshape(NW, NCHUNK, CHUNK).astype(jnp.int32)
    pos_slice = lax.slice(pos_table, (PADDING_IDX + 1, 0),
                          (L + PADDING_IDX + 1, D))
    out = _embed_ln(xf, sf, ef, word_table, pos_slice, seg_table,
                    emot_table, gamma, beta)
    return jnp.transpose(out, (2, 0, 1))
